# baseline (speedup 1.0000x reference)
import jax
import jax.numpy as jnp
from jax import lax
from jax.experimental import pallas as pl
from jax.experimental.pallas import tpu as pltpu
from jax.experimental.pallas import tpu_sc as plsc

NC, NS, L = 2, 16, 16
NQ = 4
FQ = 16
N = 50000
NPAD = 51200
RPS = NPAD // NS
IB = 160
NIB = RPS // IB
E = 800000
CB = 128
NBUF = 7
EPC_ALIGN = NS * CB * NBUF * 2
EPAD = ((E + EPC_ALIGN - 1) // EPC_ALIGN) * EPC_ALIGN
EPC = EPAD // NS
ECR = EPC // CB
NSUP = EPC // (CB * NBUF)

_mesh = plsc.VectorSubcoreMesh(
    core_axis_name="c", subcore_axis_name="s", num_cores=NC, num_subcores=NS)


def _scale16(buf, vals16, g):
    for e2 in range(L):
        v16 = jnp.broadcast_to(vals16[e2], (L,))
        r = g * L + e2
        buf[r] = buf[r] * v16


def _spmm_body(src, prev, dvec, gidx, sidx, evals, out,
               acc, cache, cidx, ridx, vbuf, msg, xbuf, dbuf, *sems):
    sem_in = sems[0:2]
    sem_g = sems[2:2 + NBUF]
    sem_sc = sems[2 + NBUF:2 + 2 * NBUF]
    sem_x = sems[2 + 2 * NBUF:4 + 2 * NBUF]
    sem_o = sems[4 + 2 * NBUF:6 + 2 * NBUF]
    sem_c = sems[6 + 2 * NBUF]
    c = lax.axis_index("c")
    s = lax.axis_index("s")

    for it in range(2):
        q = c * 2 + it

        cd = pltpu.async_copy(
            src.at[pl.ds(q * NPAD + s * RPS, RPS)],
            cache.at[pl.ds(s * RPS, RPS)], sem_c)

        def start_init_in(b, k, _q=q):
            pltpu.async_copy(
                prev.at[pl.ds(_q * NPAD + s * RPS + k * IB, IB)],
                xbuf.at[b], sem_x[b])
            pltpu.async_copy(dvec.at[pl.ds(s * RPS + k * IB, IB)],
                             dbuf.at[b], sem_x[b])

        def wait_init_in(b):
            pltpu.make_async_copy(
                prev.at[pl.ds(0, IB)], xbuf.at[b], sem_x[b]).wait()
            pltpu.make_async_copy(
                dvec.at[pl.ds(0, IB)], dbuf.at[b], sem_x[b]).wait()

        def wait_init_out(b):
            pltpu.make_async_copy(
                xbuf.at[b], acc.at[pl.ds(0, IB)], sem_o[b]).wait()

        start_init_in(0, 0)

        def init_body(tt, carry):
            for b in range(2):
                k = tt * 2 + b
                wait_init_in(b)

                def g_body(g, carry2, _b=b):
                    _scale16(xbuf.at[_b], dbuf[_b, pl.ds(g * L, L)], g)
                    return carry2

                lax.fori_loop(0, IB // L, g_body, 0)
                pltpu.async_copy(xbuf.at[b],
                                 acc.at[pl.ds(s * RPS + k * IB, IB)], sem_o[b])
                o = 1 - b

                @pl.when(k + 1 < NIB)
                def _():
                    @pl.when(k >= 1)
                    def _():
                        wait_init_out(o)
                    start_init_in(o, k + 1)
            return carry

        lax.fori_loop(0, NIB // 2, init_body, 0)
        wait_init_out(0)
        wait_init_out(1)
        cd.wait()
        plsc.subcore_barrier()

        def start_edge_in(p, t):
            roff = s * ECR + t * NBUF
            pltpu.async_copy(gidx.at[pl.ds(roff, NBUF)], cidx.at[p], sem_in[p])
            pltpu.async_copy(sidx.at[pl.ds(roff, NBUF)], ridx.at[p], sem_in[p])
            pltpu.async_copy(evals.at[pl.ds(roff, NBUF)], vbuf.at[p], sem_in[p])

        def wait_edge_in(p):
            pltpu.make_async_copy(
                gidx.at[pl.ds(0, NBUF)], cidx.at[p], sem_in[p]).wait()
            pltpu.make_async_copy(
                sidx.at[pl.ds(0, NBUF)], ridx.at[p], sem_in[p]).wait()
            pltpu.make_async_copy(
                evals.at[pl.ds(0, NBUF)], vbuf.at[p], sem_in[p]).wait()

        def wait_sc(b):
            pltpu.make_async_copy(
                msg.at[b], acc.at[pl.ds(0, CB)], sem_sc[b]).wait()

        start_edge_in(0, 0)

        def super_body(tt, carry):
            for p in range(2):
                t = tt * 2 + p

                @pl.when(t > 0)
                def _():
                    for b in range(NBUF):
                        wait_sc(b)

                wait_edge_in(p)
                gd = []
                for b in range(NBUF):
                    gd.append(pltpu.async_copy(
                        cache.at[cidx.at[p, b]], msg.at[b], sem_g[b]))

                @pl.when(t + 1 < NSUP)
                def _():
                    start_edge_in(1 - p, t + 1)

                for b in range(NBUF):
                    gd[b].wait()
                    mb = msg.at[b]
                    vb = vbuf.at[p, b]

                    def g_body(g2, carry2, _mb=mb, _vb=vb):
                        for u in range(2):
                            g = g2 * 2 + u
                            _scale16(_mb, _vb[pl.ds(g * L, L)], g)
                        return carry2

                    lax.fori_loop(0, CB // (2 * L), g_body, 0)
                    pltpu.async_copy(
                        msg.at[b], acc.at[ridx.at[p, b]], sem_sc[b], add=True)
            return carry

        lax.fori_loop(0, NSUP // 2, super_body, 0)
        for b in range(NBUF):
            wait_sc(b)
        plsc.subcore_barrier()

        pltpu.sync_copy(acc.at[pl.ds(s * RPS, RPS)],
                        out.at[pl.ds(q * NPAD + s * RPS, RPS)])


_spmm = pl.kernel(
    _spmm_body,
    out_type=jax.ShapeDtypeStruct((NQ * NPAD, FQ), jnp.float32),
    mesh=_mesh,
    scratch_types=[
        pltpu.VMEM_SHARED((NPAD, FQ), jnp.float32),
        pltpu.VMEM_SHARED((NPAD, FQ), jnp.float32),
        pltpu.VMEM((2, NBUF, CB), jnp.int32),
        pltpu.VMEM((2, NBUF, CB), jnp.int32),
        pltpu.VMEM((2, NBUF, CB), jnp.float32),
        pltpu.VMEM((NBUF, CB, FQ), jnp.float32),
        pltpu.VMEM((2, IB, FQ), jnp.float32),
        pltpu.VMEM((2, IB), jnp.float32),
    ] + [pltpu.SemaphoreType.DMA] * (2 + 2 * NBUF + 5),
    compiler_params=pltpu.CompilerParams(use_tc_tiling_on_sc=False),
)


def _relayout(x):
    xp = jnp.pad(x, ((0, NPAD - x.shape[0]), (0, 0)))
    return xp.reshape(NPAD, NQ, FQ).transpose(1, 0, 2).reshape(NQ * NPAD, FQ)


def _unlayout(x):
    return x.reshape(NQ, NPAD, FQ).transpose(1, 0, 2).reshape(NPAD, NQ * FQ)[:N]


def _bpr_body(u_ref, ii_ref, ij_ref, pi_ref, pj_ref, loss_ref, loss2_ref):
    u = u_ref[...]
    ii = ii_ref[...]
    ij = ij_ref[...]
    pi = (u * ii).sum(axis=-1)
    pj = (u * ij).sum(axis=-1)
    l2 = 0.01 * (u * u + ii * ii + ij * ij).sum(axis=-1)
    pi_ref[...] = pi
    pj_ref[...] = pj
    z = pi - pj
    sp = jnp.maximum(-z, 0.0) + jnp.log1p(jnp.exp(-jnp.abs(z)))
    loss2 = sp.mean()
    loss2_ref[0, 0] = loss2
    loss_ref[0, 0] = loss2 + l2.mean()


def _bpr_stage(u, ii, ij):
    B = u.shape[0]
    out_shape = (
        jax.ShapeDtypeStruct((B,), jnp.float32),
        jax.ShapeDtypeStruct((B,), jnp.float32),
        jax.ShapeDtypeStruct((1, 1), jnp.float32),
        jax.ShapeDtypeStruct((1, 1), jnp.float32),
    )
    pi, pj, loss, loss2 = pl.pallas_call(
        _bpr_body,
        out_shape=out_shape,
        out_specs=(
            pl.BlockSpec(memory_space=pltpu.VMEM),
            pl.BlockSpec(memory_space=pltpu.VMEM),
            pl.BlockSpec(memory_space=pltpu.SMEM),
            pl.BlockSpec(memory_space=pltpu.SMEM),
        ),
    )(u, ii, ij)
    return pi, pj, loss.reshape(()), loss2.reshape(())


def kernel(embed_user_w, embed_item_w, edge_vals, d_i, d_j, rows, cols, user, item_i, item_j):
    users0 = _relayout(embed_user_w)
    items0 = _relayout(embed_item_w)
    dip = jnp.pad(d_i.reshape(-1), (0, NPAD - N))
    djp = jnp.pad(d_j.reshape(-1), (0, NPAD - N))
    cols_p = jnp.pad(cols, (0, EPAD - E)).reshape(EPAD // CB, CB)
    rows_p = jnp.pad(rows, (0, EPAD - E)).reshape(EPAD // CB, CB)
    vals_p = jnp.pad(edge_vals, (0, EPAD - E)).reshape(EPAD // CB, CB)

    def spmm_ui(x_items, x_prev_users):
        return _spmm(x_items, x_prev_users, dip, cols_p, rows_p, vals_p)

    def spmm_iu(x_users, x_prev_items):
        return _spmm(x_users, x_prev_items, djp, rows_p, cols_p, vals_p)

    g1u = spmm_ui(items0, users0)
    g1i = spmm_iu(users0, items0)
    g2u = spmm_ui(g1i, g1u)
    g2i = spmm_iu(g1u, g1i)
    g3u = spmm_ui(g2i, g2u)
    g3i = spmm_iu(g2u, g2i)

    gcn_users = jnp.concatenate(
        (embed_user_w, _unlayout(g1u), _unlayout(g2u), _unlayout(g3u)), axis=-1)
    gcn_items = jnp.concatenate(
        (embed_item_w, _unlayout(g1i), _unlayout(g2i), _unlayout(g3i)), axis=-1)
    u = jnp.take(gcn_users, user, axis=0)
    ii = jnp.take(gcn_items, item_i, axis=0)
    ij = jnp.take(gcn_items, item_j, axis=0)
    return _bpr_stage(u, ii, ij)


# speedup vs baseline: 1.1612x; 1.1612x over previous
import jax
import jax.numpy as jnp
from jax import lax
from jax.experimental import pallas as pl
from jax.experimental.pallas import tpu as pltpu
from jax.experimental.pallas import tpu_sc as plsc

NC, NS, L = 2, 16, 16
NQ = 4
FQ = 16
N = 50000
NPAD = 51200
RPS = NPAD // NS
IB = 160
NIB = RPS // IB
E = 800000
CB = 128
NBUF = 8
EPC_ALIGN = NS * CB * NBUF
EPAD = ((E + EPC_ALIGN - 1) // EPC_ALIGN) * EPC_ALIGN
EPC = EPAD // NS
NSUP = EPC // (CB * NBUF)

_mesh = plsc.VectorSubcoreMesh(
    core_axis_name="c", subcore_axis_name="s", num_cores=NC, num_subcores=NS)


def _scale16(buf, vals16, g):
    for e2 in range(L):
        v16 = jnp.broadcast_to(vals16[e2], (L,))
        r = g * L + e2
        buf[r] = buf[r] * v16


def _spmm_body(src, prev, dvec, gidx, sidx, evals, out,
               acc, cache, cidx, ridx, vbuf, msg, xbuf, dbuf, *sems):
    sem_in = sems[0:NBUF]
    sem_g = sems[NBUF:2 * NBUF]
    sem_sc = sems[2 * NBUF:3 * NBUF]
    sem_x = sems[3 * NBUF:3 * NBUF + 2]
    sem_o = sems[3 * NBUF + 2:3 * NBUF + 4]
    sem_c = sems[3 * NBUF + 4]
    c = lax.axis_index("c")
    s = lax.axis_index("s")

    for it in range(2):
        q = c * 2 + it

        cd = pltpu.async_copy(
            src.at[pl.ds(q * NPAD + s * RPS, RPS)],
            cache.at[pl.ds(s * RPS, RPS)], sem_c)

        def start_init_in(b, k, _q=q):
            pltpu.async_copy(
                prev.at[pl.ds(_q * NPAD + s * RPS + k * IB, IB)],
                xbuf.at[b], sem_x[b])
            pltpu.async_copy(dvec.at[pl.ds(s * RPS + k * IB, IB)],
                             dbuf.at[b], sem_x[b])

        def wait_init_in(b):
            pltpu.make_async_copy(
                prev.at[pl.ds(0, IB)], xbuf.at[b], sem_x[b]).wait()
            pltpu.make_async_copy(
                dvec.at[pl.ds(0, IB)], dbuf.at[b], sem_x[b]).wait()

        def wait_init_out(b):
            pltpu.make_async_copy(
                xbuf.at[b], acc.at[pl.ds(0, IB)], sem_o[b]).wait()

        start_init_in(0, 0)

        def init_body(tt, carry):
            for b in range(2):
                k = tt * 2 + b
                wait_init_in(b)

                def g_body(g, carry2, _b=b):
                    _scale16(xbuf.at[_b], dbuf[_b, pl.ds(g * L, L)], g)
                    return carry2

                lax.fori_loop(0, IB // L, g_body, 0)
                pltpu.async_copy(xbuf.at[b],
                                 acc.at[pl.ds(s * RPS + k * IB, IB)], sem_o[b])
                o = 1 - b

                @pl.when(k + 1 < NIB)
                def _():
                    @pl.when(k >= 1)
                    def _():
                        wait_init_out(o)
                    start_init_in(o, k + 1)
            return carry

        lax.fori_loop(0, NIB // 2, init_body, 0)
        wait_init_out(0)
        wait_init_out(1)
        cd.wait()
        plsc.subcore_barrier()

        def start_edge_in(b, ch):
            eoff = s * EPC + ch * CB
            pltpu.async_copy(gidx.at[pl.ds(eoff, CB)], cidx.at[b], sem_in[b])
            pltpu.async_copy(sidx.at[pl.ds(eoff, CB)], ridx.at[b], sem_in[b])
            pltpu.async_copy(evals.at[pl.ds(eoff, CB)], vbuf.at[b], sem_in[b])

        def wait_edge_in(b):
            pltpu.make_async_copy(
                gidx.at[pl.ds(0, CB)], cidx.at[b], sem_in[b]).wait()
            pltpu.make_async_copy(
                sidx.at[pl.ds(0, CB)], ridx.at[b], sem_in[b]).wait()
            pltpu.make_async_copy(
                evals.at[pl.ds(0, CB)], vbuf.at[b], sem_in[b]).wait()

        for b in range(NBUF):
            start_edge_in(b, b)

        def super_body(t, carry):
            gd, sd = [], []
            for b in range(NBUF):
                wait_edge_in(b)
                gd.append(pltpu.async_copy(
                    cache.at[cidx.at[b]], msg.at[b], sem_g[b]))
            for b in range(NBUF):
                gd[b].wait()

                def g_body(g, carry2, _b=b):
                    _scale16(msg.at[_b], vbuf[_b, pl.ds(g * L, L)], g)
                    return carry2

                lax.fori_loop(0, CB // L, g_body, 0)
                sd.append(pltpu.async_copy(
                    msg.at[b], acc.at[ridx.at[b]], sem_sc[b], add=True))
            for b in range(NBUF):
                sd[b].wait()

                @pl.when(t + 1 < NSUP)
                def _(_b=b):
                    start_edge_in(_b, (t + 1) * NBUF + _b)
            return carry

        lax.fori_loop(0, NSUP, super_body, 0)
        plsc.subcore_barrier()

        pltpu.sync_copy(acc.at[pl.ds(s * RPS, RPS)],
                        out.at[pl.ds(q * NPAD + s * RPS, RPS)])


_spmm = pl.kernel(
    _spmm_body,
    out_type=jax.ShapeDtypeStruct((NQ * NPAD, FQ), jnp.float32),
    mesh=_mesh,
    scratch_types=[
        pltpu.VMEM_SHARED((NPAD, FQ), jnp.float32),
        pltpu.VMEM_SHARED((NPAD, FQ), jnp.float32),
        pltpu.VMEM((NBUF, CB), jnp.int32),
        pltpu.VMEM((NBUF, CB), jnp.int32),
        pltpu.VMEM((NBUF, CB), jnp.float32),
        pltpu.VMEM((NBUF, CB, FQ), jnp.float32),
        pltpu.VMEM((2, IB, FQ), jnp.float32),
        pltpu.VMEM((2, IB), jnp.float32),
    ] + [pltpu.SemaphoreType.DMA] * (3 * NBUF + 5),
    compiler_params=pltpu.CompilerParams(use_tc_tiling_on_sc=False),
)


def _relayout(x):
    xp = jnp.pad(x, ((0, NPAD - x.shape[0]), (0, 0)))
    return xp.reshape(NPAD, NQ, FQ).transpose(1, 0, 2).reshape(NQ * NPAD, FQ)


def _unlayout(x):
    return x.reshape(NQ, NPAD, FQ).transpose(1, 0, 2).reshape(NPAD, NQ * FQ)[:N]


def _bpr_body(u_ref, ii_ref, ij_ref, pi_ref, pj_ref, loss_ref, loss2_ref):
    u = u_ref[...]
    ii = ii_ref[...]
    ij = ij_ref[...]
    pi = (u * ii).sum(axis=-1)
    pj = (u * ij).sum(axis=-1)
    l2 = 0.01 * (u * u + ii * ii + ij * ij).sum(axis=-1)
    pi_ref[...] = pi
    pj_ref[...] = pj
    z = pi - pj
    sp = jnp.maximum(-z, 0.0) + jnp.log1p(jnp.exp(-jnp.abs(z)))
    loss2 = sp.mean()
    loss2_ref[0, 0] = loss2
    loss_ref[0, 0] = loss2 + l2.mean()


def _bpr_stage(u, ii, ij):
    B = u.shape[0]
    out_shape = (
        jax.ShapeDtypeStruct((B,), jnp.float32),
        jax.ShapeDtypeStruct((B,), jnp.float32),
        jax.ShapeDtypeStruct((1, 1), jnp.float32),
        jax.ShapeDtypeStruct((1, 1), jnp.float32),
    )
    pi, pj, loss, loss2 = pl.pallas_call(
        _bpr_body,
        out_shape=out_shape,
        out_specs=(
            pl.BlockSpec(memory_space=pltpu.VMEM),
            pl.BlockSpec(memory_space=pltpu.VMEM),
            pl.BlockSpec(memory_space=pltpu.SMEM),
            pl.BlockSpec(memory_space=pltpu.SMEM),
        ),
    )(u, ii, ij)
    return pi, pj, loss.reshape(()), loss2.reshape(())


def kernel(embed_user_w, embed_item_w, edge_vals, d_i, d_j, rows, cols, user, item_i, item_j):
    users0 = _relayout(embed_user_w)
    items0 = _relayout(embed_item_w)
    dip = jnp.pad(d_i.reshape(-1), (0, NPAD - N))
    djp = jnp.pad(d_j.reshape(-1), (0, NPAD - N))
    cols_p = jnp.pad(cols, (0, EPAD - E))
    rows_p = jnp.pad(rows, (0, EPAD - E))
    vals_p = jnp.pad(edge_vals, (0, EPAD - E))

    def spmm_ui(x_items, x_prev_users):
        return _spmm(x_items, x_prev_users, dip, cols_p, rows_p, vals_p)

    def spmm_iu(x_users, x_prev_items):
        return _spmm(x_users, x_prev_items, djp, rows_p, cols_p, vals_p)

    g1u = spmm_ui(items0, users0)
    g1i = spmm_iu(users0, items0)
    g2u = spmm_ui(g1i, g1u)
    g2i = spmm_iu(g1u, g1i)
    g3u = spmm_ui(g2i, g2u)
    g3i = spmm_iu(g2u, g2i)

    gcn_users = jnp.concatenate(
        (embed_user_w, _unlayout(g1u), _unlayout(g2u), _unlayout(g3u)), axis=-1)
    gcn_items = jnp.concatenate(
        (embed_item_w, _unlayout(g1i), _unlayout(g2i), _unlayout(g3i)), axis=-1)
    u = jnp.take(gcn_users, user, axis=0)
    ii = jnp.take(gcn_items, item_i, axis=0)
    ij = jnp.take(gcn_items, item_j, axis=0)
    return _bpr_stage(u, ii, ij)
